# Initial kernel scaffold; baseline (speedup 1.0000x reference)
import functools

import jax
import jax.numpy as jnp
from jax import lax
from jax.experimental import pallas as pl
from jax.experimental.pallas import tpu as pltpu
from jax.experimental.pallas import tpu_sc as plsc

NC = 2
NS = 16
LANES = 16
B = 80


def _scale_rows(gath, lv, n_rows, d):
    @pl.loop(0, n_rows // LANES)
    def _(g):
        for cc in range(LANES):
            r = g * LANES + cc
            lvs = plsc.load_gather(lv, [jnp.full((LANES,), r, jnp.int32)])
            for j in range(d // LANES):
                sl = pl.ds(j * LANES, LANES)
                gath[r, sl] = gath[r, sl] * lvs


def _zero_fill(zbuf, zrows, d):
    @pl.loop(0, zrows)
    def _(i):
        for j in range(d // LANES):
            zbuf[i, pl.ds(j * LANES, LANES)] = jnp.zeros((LANES,), jnp.float32)


@functools.lru_cache(maxsize=None)
def _spmm_coeffs(n, d, e):
    ew = e // (NC * NS)
    n_chunks = ew // B
    rows_per_sub = n // NS
    zrows = 125
    mesh = plsc.VectorSubcoreMesh(core_axis_name="c", subcore_axis_name="s")

    @functools.partial(
        pl.kernel,
        mesh=mesh,
        out_type=jax.ShapeDtypeStruct((NC, n, d), jnp.float32),
        scratch_types=[
            pltpu.VMEM_SHARED((n, d), jnp.float32),
            pltpu.VMEM((B, d), jnp.float32),
            pltpu.VMEM((B,), jnp.int32),
            pltpu.VMEM((B,), jnp.int32),
            pltpu.VMEM((B,), jnp.float32),
            pltpu.VMEM((125, d), jnp.float32),
        ],
    )
    def k(x_hbm, row_hbm, col_hbm, lv_hbm, out_hbm, acc, gath, ridx, cidx, lv,
          zbuf):
        c = lax.axis_index("c")
        s = lax.axis_index("s")
        wid = s * NC + c
        _zero_fill(zbuf, zrows, d)
        for t in range(rows_per_sub // zrows):
            pltpu.sync_copy(zbuf, acc.at[pl.ds(s * rows_per_sub + t * zrows,
                                               zrows)])
        plsc.subcore_barrier()

        base = wid * ew

        @pl.loop(0, n_chunks)
        def _(i):
            off = base + i * B
            pltpu.sync_copy(row_hbm.at[pl.ds(off, B)], ridx)
            pltpu.sync_copy(col_hbm.at[pl.ds(off, B)], cidx)
            pltpu.sync_copy(lv_hbm.at[pl.ds(off, B)], lv)
            pltpu.sync_copy(x_hbm.at[ridx], gath)
            _scale_rows(gath, lv, B, d)
            pltpu.sync_copy(gath, acc.at[cidx], add=True)

        plsc.subcore_barrier()
        for t in range(rows_per_sub // zrows):
            r0 = s * rows_per_sub + t * zrows
            pltpu.sync_copy(acc.at[pl.ds(r0, zrows)],
                            out_hbm.at[c, pl.ds(r0, zrows)])

    return k


@functools.lru_cache(maxsize=None)
def _spmm_out(n, d, e, kf_total):
    es = e // NS
    n_chunks = es // B
    rows_per_sub = n // NS
    zrows = 125
    sweeps = kf_total // NC
    mesh = plsc.VectorSubcoreMesh(core_axis_name="c", subcore_axis_name="s")

    @functools.partial(
        pl.kernel,
        mesh=mesh,
        out_type=jax.ShapeDtypeStruct((kf_total, n, d), jnp.float32),
        scratch_types=[
            pltpu.VMEM_SHARED((n, d), jnp.float32),
            pltpu.VMEM((B, d), jnp.float32),
            pltpu.VMEM((B,), jnp.int32),
            pltpu.VMEM((B,), jnp.int32),
            pltpu.VMEM((B,), jnp.float32),
            pltpu.VMEM((125, d), jnp.float32),
        ],
    )
    def k(m_hbm, row_hbm, col_hbm, lv_hbm, out_hbm, acc, gath, ridx, cidx, lv,
          zbuf):
        c = lax.axis_index("c")
        s = lax.axis_index("s")
        _zero_fill(zbuf, zrows, d)
        base = s * es

        for t in range(sweeps):
            kf = c * sweeps + t
            for tz in range(rows_per_sub // zrows):
                pltpu.sync_copy(
                    zbuf, acc.at[pl.ds(s * rows_per_sub + tz * zrows, zrows)])
            plsc.subcore_barrier()

            @pl.loop(0, n_chunks)
            def _(i):
                off = base + i * B
                pltpu.sync_copy(row_hbm.at[pl.ds(off, B)], ridx)
                pltpu.sync_copy(col_hbm.at[pl.ds(off, B)], cidx)
                pltpu.sync_copy(lv_hbm.at[pl.ds(off, B)], lv)

                @pl.loop(0, B // LANES)
                def _(g):
                    sl = pl.ds(g * LANES, LANES)
                    cidx[sl] = cidx[sl] + jnp.full((LANES,), kf * n, jnp.int32)

                pltpu.sync_copy(m_hbm.at[cidx], gath)
                _scale_rows(gath, lv, B, d)
                pltpu.sync_copy(gath, acc.at[ridx], add=True)

            plsc.subcore_barrier()
            for tz in range(rows_per_sub // zrows):
                r0 = s * rows_per_sub + tz * zrows
                pltpu.sync_copy(acc.at[pl.ds(r0, zrows)],
                                out_hbm.at[kf, pl.ds(r0, zrows)])
            plsc.subcore_barrier()

    return k


@functools.lru_cache(maxsize=None)
def _weights_tc(n, d, kf, p_ord, nb):
    def body(cp0_ref, cp1_ref, eig_ref, a2_ref, m_ref):
        csum = cp0_ref[...] + cp1_ref[...]
        ex = jnp.exp(-eig_ref[...])
        pw = jnp.ones_like(ex)
        w = [None] * kf
        for pp in range(p_ord):
            for kk in range(kf):
                term = pw * a2_ref[kk, pp, :][None, :]
                w[kk] = term if pp == 0 else w[kk] + term
            pw = pw * ex
        for kk in range(kf):
            m_ref[kk] = csum * w[kk]

    return pl.pallas_call(
        body,
        grid=(n // nb,),
        in_specs=[
            pl.BlockSpec((nb, d), lambda i: (i, 0)),
            pl.BlockSpec((nb, d), lambda i: (i, 0)),
            pl.BlockSpec((nb, 1), lambda i: (i, 0)),
            pl.BlockSpec((kf, p_ord, d), lambda i: (0, 0, 0)),
        ],
        out_specs=pl.BlockSpec((kf, nb, d), lambda i: (0, i, 0)),
        out_shape=jax.ShapeDtypeStruct((kf, n, d), jnp.float32),
    )


def kernel(x, L_i, L_v, node_attr_eig, alpha):
    n, d = x.shape
    e = L_v.shape[0]
    _, kf, p_ord = alpha.shape
    row = L_i[0].astype(jnp.int32)
    col = L_i[1].astype(jnp.int32)
    lv = L_v.astype(jnp.float32)

    c_part = _spmm_coeffs(n, d, e)(x, row, col, lv)
    a2 = jnp.transpose(alpha, (1, 2, 0))
    m = _weights_tc(n, d, kf, p_ord, 1000)(
        c_part[0], c_part[1], node_attr_eig.reshape(n, 1), a2)
    outk = _spmm_out(n, d, e, kf)(m.reshape(kf * n, d), row, col, lv)
    return jnp.transpose(outk, (1, 2, 0))


# speedup vs baseline: 2.9975x; 2.9975x over previous
import dataclasses
import functools

import jax
import jax.numpy as jnp
from jax import lax
from jax.experimental import pallas as pl
from jax.experimental.pallas import tpu as pltpu
from jax.experimental.pallas import tpu_sc as plsc

NC = 2
NS = 16
LANES = 16
B = 80


def _sc_compiler_params():
    cp = pltpu.CompilerParams()
    if "needs_layout_passes" in pltpu.CompilerParams.__dataclass_fields__:
        cp = dataclasses.replace(cp, needs_layout_passes=False)
    return cp


def _scale_rows(gath, lvb, n_rows, d):
    @pl.loop(0, n_rows)
    def _(r):
        lvs = lvb[r]
        for j in range(d // LANES):
            sl = pl.ds(j * LANES, LANES)
            gath[r, sl] = gath[r, sl] * lvs


def _zero_fill(zbuf, zrows, d):
    @pl.loop(0, zrows)
    def _(i):
        for j in range(d // LANES):
            zbuf[i, pl.ds(j * LANES, LANES)] = jnp.zeros((LANES,), jnp.float32)


def _pad_rows(n):
    return ((n + NS * 128 - 1) // (NS * 128)) * (NS * 128)


@functools.lru_cache(maxsize=None)
def _spmm_coeffs(n, d, e):
    ew = e // (NC * NS)
    n_chunks = ew // B
    npad = _pad_rows(n)
    rows_per_sub = npad // NS
    zrows = 128
    mesh = plsc.VectorSubcoreMesh(core_axis_name="c", subcore_axis_name="s")

    @functools.partial(
        pl.kernel,
        mesh=mesh,
        compiler_params=_sc_compiler_params(),
        out_type=jax.ShapeDtypeStruct((NC, npad, d), jnp.float32),
        scratch_types=[
            pltpu.VMEM_SHARED((npad, d), jnp.float32),
            pltpu.VMEM((B, d), jnp.float32),
            pltpu.VMEM((B,), jnp.int32),
            pltpu.VMEM((B,), jnp.int32),
            pltpu.VMEM((B, LANES), jnp.float32),
            pltpu.VMEM((zrows, d), jnp.float32),
        ],
    )
    def k(x_hbm, row_hbm, col_hbm, lvx_hbm, out_hbm, acc, gath, ridx, cidx,
          lvb, zbuf):
        c = lax.axis_index("c")
        s = lax.axis_index("s")
        wid = s * NC + c
        _zero_fill(zbuf, zrows, d)
        for t in range(rows_per_sub // zrows):
            pltpu.sync_copy(zbuf, acc.at[pl.ds(s * rows_per_sub + t * zrows,
                                               zrows)])
        plsc.subcore_barrier()

        base = wid * ew

        @pl.loop(0, n_chunks)
        def _(i):
            off = base + i * B
            pltpu.sync_copy(row_hbm.at[pl.ds(off, B)], ridx)
            pltpu.sync_copy(col_hbm.at[pl.ds(off, B)], cidx)
            pltpu.sync_copy(lvx_hbm.at[pl.ds(off, B)], lvb)
            pltpu.sync_copy(x_hbm.at[ridx], gath)
            _scale_rows(gath, lvb, B, d)
            pltpu.sync_copy(gath, acc.at[cidx], add=True)

        plsc.subcore_barrier()
        for t in range(rows_per_sub // zrows):
            r0 = s * rows_per_sub + t * zrows
            pltpu.sync_copy(acc.at[pl.ds(r0, zrows)],
                            out_hbm.at[c, pl.ds(r0, zrows)])

    return k


@functools.lru_cache(maxsize=None)
def _spmm_out(n, d, e, kf_total):
    es = e // NS
    n_chunks = es // B
    npad = _pad_rows(n)
    rows_per_sub = npad // NS
    zrows = 128
    sweeps = kf_total // NC
    mesh = plsc.VectorSubcoreMesh(core_axis_name="c", subcore_axis_name="s")

    @functools.partial(
        pl.kernel,
        mesh=mesh,
        compiler_params=_sc_compiler_params(),
        out_type=jax.ShapeDtypeStruct((kf_total, npad, d), jnp.float32),
        scratch_types=[
            pltpu.VMEM_SHARED((npad, d), jnp.float32),
            pltpu.VMEM((B, d), jnp.float32),
            pltpu.VMEM((B,), jnp.int32),
            pltpu.VMEM((B,), jnp.int32),
            pltpu.VMEM((B, LANES), jnp.float32),
            pltpu.VMEM((zrows, d), jnp.float32),
        ],
    )
    def k(m_hbm, row_hbm, col_hbm, lvx_hbm, out_hbm, acc, gath, ridx, cidx,
          lvb, zbuf):
        c = lax.axis_index("c")
        s = lax.axis_index("s")
        _zero_fill(zbuf, zrows, d)
        base = s * es

        for t in range(sweeps):
            kf = c * sweeps + t
            for tz in range(rows_per_sub // zrows):
                pltpu.sync_copy(
                    zbuf, acc.at[pl.ds(s * rows_per_sub + tz * zrows, zrows)])
            plsc.subcore_barrier()

            @pl.loop(0, n_chunks)
            def _(i):
                off = base + i * B
                pltpu.sync_copy(row_hbm.at[pl.ds(off, B)], ridx)
                pltpu.sync_copy(col_hbm.at[pl.ds(off, B)], cidx)
                pltpu.sync_copy(lvx_hbm.at[pl.ds(off, B)], lvb)

                @pl.loop(0, B // LANES)
                def _(g):
                    sl = pl.ds(g * LANES, LANES)
                    cidx[sl] = cidx[sl] + jnp.full((LANES,), kf * n, jnp.int32)

                pltpu.sync_copy(m_hbm.at[cidx], gath)
                _scale_rows(gath, lvb, B, d)
                pltpu.sync_copy(gath, acc.at[ridx], add=True)

            plsc.subcore_barrier()
            for tz in range(rows_per_sub // zrows):
                r0 = s * rows_per_sub + tz * zrows
                pltpu.sync_copy(acc.at[pl.ds(r0, zrows)],
                                out_hbm.at[kf, pl.ds(r0, zrows)])
            plsc.subcore_barrier()

    return k


@functools.lru_cache(maxsize=None)
def _weights_tc(n, d, kf, p_ord, nb):
    def body(cp0_ref, cp1_ref, eig_ref, a2_ref, m_ref):
        csum = cp0_ref[...] + cp1_ref[...]
        ex = jnp.exp(-eig_ref[...])
        pw = jnp.ones_like(ex)
        w = [None] * kf
        for pp in range(p_ord):
            for kk in range(kf):
                term = pw * a2_ref[kk, pp, :][None, :]
                w[kk] = term if pp == 0 else w[kk] + term
            pw = pw * ex
        for kk in range(kf):
            m_ref[kk] = csum * w[kk]

    return pl.pallas_call(
        body,
        grid=(n // nb,),
        in_specs=[
            pl.BlockSpec((nb, d), lambda i: (i, 0)),
            pl.BlockSpec((nb, d), lambda i: (i, 0)),
            pl.BlockSpec((nb, 1), lambda i: (i, 0)),
            pl.BlockSpec((kf, p_ord, d), lambda i: (0, 0, 0)),
        ],
        out_specs=pl.BlockSpec((kf, nb, d), lambda i: (0, i, 0)),
        out_shape=jax.ShapeDtypeStruct((kf, n, d), jnp.float32),
    )


def kernel(x, L_i, L_v, node_attr_eig, alpha):
    n, d = x.shape
    e = L_v.shape[0]
    _, kf, p_ord = alpha.shape
    row = L_i[0].astype(jnp.int32)
    col = L_i[1].astype(jnp.int32)
    lvx = jnp.broadcast_to(L_v.astype(jnp.float32)[:, None], (e, LANES))

    c_part = _spmm_coeffs(n, d, e)(x, row, col, lvx)[:, :n]
    a2 = jnp.transpose(alpha, (1, 2, 0))
    m = _weights_tc(n, d, kf, p_ord, 1000)(
        c_part[0], c_part[1], node_attr_eig.reshape(n, 1), a2)
    outk = _spmm_out(n, d, e, kf)(m.reshape(kf * n, d), row, col, lvx)
    return jnp.transpose(outk[:, :n], (1, 2, 0))


# speedup vs baseline: 3.6691x; 1.2241x over previous
import dataclasses
import functools

import jax
import jax.numpy as jnp
from jax import lax
from jax.experimental import pallas as pl
from jax.experimental.pallas import tpu as pltpu
from jax.experimental.pallas import tpu_sc as plsc

NC = 2
NS = 16
LANES = 16
B = 128


def _split_chunks(per_worker):
    n_full = per_worker // B
    tail = per_worker - n_full * B
    if tail % 8:
        raise ValueError(f"edge split {per_worker} not 8-aligned with B={B}")
    return n_full, tail


def _sc_compiler_params():
    cp = pltpu.CompilerParams()
    if "needs_layout_passes" in pltpu.CompilerParams.__dataclass_fields__:
        cp = dataclasses.replace(cp, needs_layout_passes=False)
    return cp


def _scale_rows(gath, lvb, n_rows, d):
    @pl.loop(0, n_rows)
    def _(r):
        lvs = lvb[r]
        for j in range(d // LANES):
            sl = pl.ds(j * LANES, LANES)
            gath[r, sl] = gath[r, sl] * lvs


def _zero_fill(zbuf, zrows, d):
    @pl.loop(0, zrows)
    def _(i):
        for j in range(d // LANES):
            zbuf[i, pl.ds(j * LANES, LANES)] = jnp.zeros((LANES,), jnp.float32)


def _pad_rows(n):
    return ((n + NS * 128 - 1) // (NS * 128)) * (NS * 128)


@functools.lru_cache(maxsize=None)
def _spmm_coeffs(n, d, e):
    ew = e // (NC * NS)
    n_full, tail = _split_chunks(ew)
    npad = _pad_rows(n)
    rows_per_sub = npad // NS
    zrows = 64
    mesh = plsc.VectorSubcoreMesh(core_axis_name="c", subcore_axis_name="s")

    scratch = [
        pltpu.VMEM_SHARED((npad, d), jnp.float32),
        pltpu.VMEM((B, d), jnp.float32),
        pltpu.VMEM((B,), jnp.int32),
        pltpu.VMEM((B,), jnp.int32),
        pltpu.VMEM((B, LANES), jnp.float32),
    ]
    if tail:
        scratch += [
            pltpu.VMEM((tail, d), jnp.float32),
            pltpu.VMEM((tail,), jnp.int32),
            pltpu.VMEM((tail,), jnp.int32),
            pltpu.VMEM((tail, LANES), jnp.float32),
        ]

    @functools.partial(
        pl.kernel,
        mesh=mesh,
        compiler_params=_sc_compiler_params(),
        out_type=jax.ShapeDtypeStruct((NC, npad, d), jnp.float32),
        scratch_types=scratch,
    )
    def k(x_hbm, row_hbm, col_hbm, lvx_hbm, out_hbm, acc, gath, ridx, cidx,
          lvb, *tbufs):
        c = lax.axis_index("c")
        s = lax.axis_index("s")
        wid = s * NC + c
        _zero_fill(gath, zrows, d)
        for t in range(rows_per_sub // zrows):
            pltpu.sync_copy(gath.at[pl.ds(0, zrows)],
                            acc.at[pl.ds(s * rows_per_sub + t * zrows,
                                         zrows)])
        plsc.subcore_barrier()

        base = wid * ew

        @pl.loop(0, n_full)
        def _(i):
            off = base + i * B
            pltpu.sync_copy(row_hbm.at[pl.ds(off, B)], ridx)
            pltpu.sync_copy(col_hbm.at[pl.ds(off, B)], cidx)
            pltpu.sync_copy(lvx_hbm.at[pl.ds(off, B)], lvb)
            pltpu.sync_copy(x_hbm.at[ridx], gath)
            _scale_rows(gath, lvb, B, d)
            pltpu.sync_copy(gath, acc.at[cidx], add=True)

        if tail:
            gath_t, ridx_t, cidx_t, lvb_t = tbufs
            off = base + n_full * B
            pltpu.sync_copy(row_hbm.at[pl.ds(off, tail)], ridx_t)
            pltpu.sync_copy(col_hbm.at[pl.ds(off, tail)], cidx_t)
            pltpu.sync_copy(lvx_hbm.at[pl.ds(off, tail)], lvb_t)
            pltpu.sync_copy(x_hbm.at[ridx_t], gath_t)
            _scale_rows(gath_t, lvb_t, tail, d)
            pltpu.sync_copy(gath_t, acc.at[cidx_t], add=True)

        plsc.subcore_barrier()
        for t in range(rows_per_sub // zrows):
            r0 = s * rows_per_sub + t * zrows
            pltpu.sync_copy(acc.at[pl.ds(r0, zrows)],
                            out_hbm.at[c, pl.ds(r0, zrows)])

    return k


@functools.lru_cache(maxsize=None)
def _spmm_out(n, d, e, kf_total):
    es = e // NS
    n_full, tail = _split_chunks(es)
    npad = _pad_rows(n)
    rows_per_sub = npad // NS
    zrows = 64
    sweeps = kf_total // NC
    mesh = plsc.VectorSubcoreMesh(core_axis_name="c", subcore_axis_name="s")

    scratch = [
        pltpu.VMEM_SHARED((npad, d), jnp.float32),
        pltpu.VMEM((B, d), jnp.float32),
        pltpu.VMEM((B,), jnp.int32),
        pltpu.VMEM((B,), jnp.int32),
        pltpu.VMEM((B, LANES), jnp.float32),
    ]
    if tail:
        scratch += [
            pltpu.VMEM((tail, d), jnp.float32),
            pltpu.VMEM((tail,), jnp.int32),
            pltpu.VMEM((tail,), jnp.int32),
            pltpu.VMEM((tail, LANES), jnp.float32),
        ]

    @functools.partial(
        pl.kernel,
        mesh=mesh,
        compiler_params=_sc_compiler_params(),
        out_type=jax.ShapeDtypeStruct((kf_total, npad, d), jnp.float32),
        scratch_types=scratch,
    )
    def k(m_hbm, row_hbm, col_hbm, lvx_hbm, out_hbm, acc, gath, ridx, cidx,
          lvb, *tbufs):
        c = lax.axis_index("c")
        s = lax.axis_index("s")
        base = s * es

        for t in range(sweeps):
            kf = c * sweeps + t
            _zero_fill(gath, zrows, d)
            for tz in range(rows_per_sub // zrows):
                pltpu.sync_copy(
                    gath.at[pl.ds(0, zrows)],
                    acc.at[pl.ds(s * rows_per_sub + tz * zrows, zrows)])
            plsc.subcore_barrier()

            @pl.loop(0, n_full)
            def _(i):
                off = base + i * B
                pltpu.sync_copy(row_hbm.at[pl.ds(off, B)], ridx)
                pltpu.sync_copy(col_hbm.at[pl.ds(off, B)], cidx)
                pltpu.sync_copy(lvx_hbm.at[pl.ds(off, B)], lvb)

                @pl.loop(0, B // LANES)
                def _(g):
                    sl = pl.ds(g * LANES, LANES)
                    cidx[sl] = cidx[sl] + jnp.full((LANES,), kf * n, jnp.int32)

                pltpu.sync_copy(m_hbm.at[cidx], gath)
                _scale_rows(gath, lvb, B, d)
                pltpu.sync_copy(gath, acc.at[ridx], add=True)

            if tail:
                gath_t, ridx_t, cidx_t, lvb_t = tbufs
                off = base + n_full * B
                pltpu.sync_copy(row_hbm.at[pl.ds(off, tail)], ridx_t)
                pltpu.sync_copy(col_hbm.at[pl.ds(off, tail)], cidx_t)
                pltpu.sync_copy(lvx_hbm.at[pl.ds(off, tail)], lvb_t)

                @pl.loop(0, tail // LANES)
                def _(g):
                    sl = pl.ds(g * LANES, LANES)
                    cidx_t[sl] = cidx_t[sl] + jnp.full((LANES,), kf * n,
                                                       jnp.int32)

                pltpu.sync_copy(m_hbm.at[cidx_t], gath_t)
                _scale_rows(gath_t, lvb_t, tail, d)
                pltpu.sync_copy(gath_t, acc.at[ridx_t], add=True)

            plsc.subcore_barrier()
            for tz in range(rows_per_sub // zrows):
                r0 = s * rows_per_sub + tz * zrows
                pltpu.sync_copy(acc.at[pl.ds(r0, zrows)],
                                out_hbm.at[kf, pl.ds(r0, zrows)])
            plsc.subcore_barrier()

    return k


@functools.lru_cache(maxsize=None)
def _weights_tc(n, d, kf, p_ord, nb):
    def body(cp0_ref, cp1_ref, eig_ref, a2_ref, m_ref):
        csum = cp0_ref[...] + cp1_ref[...]
        ex = jnp.exp(-eig_ref[...])
        pw = jnp.ones_like(ex)
        w = [None] * kf
        for pp in range(p_ord):
            for kk in range(kf):
                term = pw * a2_ref[kk, pp, :][None, :]
                w[kk] = term if pp == 0 else w[kk] + term
            pw = pw * ex
        for kk in range(kf):
            m_ref[kk] = csum * w[kk]

    return pl.pallas_call(
        body,
        grid=(n // nb,),
        in_specs=[
            pl.BlockSpec((nb, d), lambda i: (i, 0)),
            pl.BlockSpec((nb, d), lambda i: (i, 0)),
            pl.BlockSpec((nb, 1), lambda i: (i, 0)),
            pl.BlockSpec((kf, p_ord, d), lambda i: (0, 0, 0)),
        ],
        out_specs=pl.BlockSpec((kf, nb, d), lambda i: (0, i, 0)),
        out_shape=jax.ShapeDtypeStruct((kf, n, d), jnp.float32),
    )


def kernel(x, L_i, L_v, node_attr_eig, alpha):
    n, d = x.shape
    e = L_v.shape[0]
    _, kf, p_ord = alpha.shape
    row = L_i[0].astype(jnp.int32)
    col = L_i[1].astype(jnp.int32)
    lvx = jnp.broadcast_to(L_v.astype(jnp.float32)[:, None], (e, LANES))

    c_part = _spmm_coeffs(n, d, e)(x, row, col, lvx)[:, :n]
    a2 = jnp.transpose(alpha, (1, 2, 0))
    m = _weights_tc(n, d, kf, p_ord, 1000)(
        c_part[0], c_part[1], node_attr_eig.reshape(n, 1), a2)
    outk = _spmm_out(n, d, e, kf)(m.reshape(kf * n, d), row, col, lvx)
    return jnp.transpose(outk[:, :n], (1, 2, 0))


# speedup vs baseline: 4.3615x; 1.1887x over previous
import dataclasses
import functools

import jax
import jax.numpy as jnp
from jax import lax
from jax.experimental import pallas as pl
from jax.experimental.pallas import tpu as pltpu
from jax.experimental.pallas import tpu_sc as plsc

NC = 2
NS = 16
LANES = 16
B = 128


def _split_chunks(per_worker):
    n_full = per_worker // B
    tail = per_worker - n_full * B
    if tail % 8:
        raise ValueError(f"edge split {per_worker} not 8-aligned with B={B}")
    return n_full, tail


def _sc_compiler_params():
    cp = pltpu.CompilerParams()
    if "needs_layout_passes" in pltpu.CompilerParams.__dataclass_fields__:
        cp = dataclasses.replace(cp, needs_layout_passes=False)
    return cp


def _scale_rows(gath, lvb, n_rows, d):
    @pl.loop(0, n_rows)
    def _(r):
        lvs = lvb[r]
        for j in range(d // LANES):
            sl = pl.ds(j * LANES, LANES)
            gath[r, sl] = gath[r, sl] * lvs


def _zero_fill(zbuf, zrows, d):
    @pl.loop(0, zrows)
    def _(i):
        for j in range(d // LANES):
            zbuf[i, pl.ds(j * LANES, LANES)] = jnp.zeros((LANES,), jnp.float32)


def _pad_rows(n):
    return ((n + NS * 128 - 1) // (NS * 128)) * (NS * 128)


@functools.lru_cache(maxsize=None)
def _spmm_coeffs(n, d, e):
    ew = e // (NC * NS)
    n_full, tail = _split_chunks(ew)
    npad = _pad_rows(n)
    rows_per_sub = npad // NS
    zrows = 64
    mesh = plsc.VectorSubcoreMesh(core_axis_name="c", subcore_axis_name="s")

    scratch = [
        pltpu.VMEM_SHARED((npad, d), jnp.float32),
        pltpu.VMEM((B, d), jnp.float32),
        pltpu.VMEM((B,), jnp.int32),
        pltpu.VMEM((B,), jnp.int32),
        pltpu.VMEM((B, LANES), jnp.float32),
        pltpu.SemaphoreType.DMA,
    ]
    if tail:
        scratch += [
            pltpu.VMEM((tail, d), jnp.float32),
            pltpu.VMEM((tail,), jnp.int32),
            pltpu.VMEM((tail,), jnp.int32),
            pltpu.VMEM((tail, LANES), jnp.float32),
        ]

    @functools.partial(
        pl.kernel,
        mesh=mesh,
        compiler_params=_sc_compiler_params(),
        out_type=jax.ShapeDtypeStruct((NC, npad, d), jnp.float32),
        scratch_types=scratch,
    )
    def k(x_hbm, row_hbm, col_hbm, lvx_hbm, out_hbm, acc, gath, ridx, cidx,
          lvb, sem, *tbufs):
        c = lax.axis_index("c")
        s = lax.axis_index("s")
        wid = s * NC + c
        _zero_fill(gath, zrows, d)
        for t in range(rows_per_sub // zrows):
            pltpu.sync_copy(gath.at[pl.ds(0, zrows)],
                            acc.at[pl.ds(s * rows_per_sub + t * zrows,
                                         zrows)])
        plsc.subcore_barrier()

        base = wid * ew

        @pl.loop(0, n_full)
        def _(i):
            off = base + i * B
            h1 = pltpu.async_copy(row_hbm.at[pl.ds(off, B)], ridx, sem)
            h2 = pltpu.async_copy(col_hbm.at[pl.ds(off, B)], cidx, sem)
            h3 = pltpu.async_copy(lvx_hbm.at[pl.ds(off, B)], lvb, sem)
            h1.wait()
            h2.wait()
            h3.wait()
            pltpu.sync_copy(x_hbm.at[ridx], gath)
            _scale_rows(gath, lvb, B, d)
            pltpu.sync_copy(gath, acc.at[cidx], add=True)

        if tail:
            gath_t, ridx_t, cidx_t, lvb_t = tbufs
            off = base + n_full * B
            pltpu.sync_copy(row_hbm.at[pl.ds(off, tail)], ridx_t)
            pltpu.sync_copy(col_hbm.at[pl.ds(off, tail)], cidx_t)
            pltpu.sync_copy(lvx_hbm.at[pl.ds(off, tail)], lvb_t)
            pltpu.sync_copy(x_hbm.at[ridx_t], gath_t)
            _scale_rows(gath_t, lvb_t, tail, d)
            pltpu.sync_copy(gath_t, acc.at[cidx_t], add=True)

        plsc.subcore_barrier()
        for t in range(rows_per_sub // zrows):
            r0 = s * rows_per_sub + t * zrows
            pltpu.sync_copy(acc.at[pl.ds(r0, zrows)],
                            out_hbm.at[c, pl.ds(r0, zrows)])

    return k


@functools.lru_cache(maxsize=None)
def _spmm_out(n, d, e, kf_total):
    es = e // NS
    n_full, tail = _split_chunks(es)
    npad = _pad_rows(n)
    rows_per_sub = npad // NS
    zrows = 64
    sweeps = kf_total // NC
    mesh = plsc.VectorSubcoreMesh(core_axis_name="c", subcore_axis_name="s")

    scratch = [
        pltpu.VMEM_SHARED((npad, d), jnp.float32),
        pltpu.VMEM((B, d), jnp.float32),
        pltpu.VMEM((B,), jnp.int32),
        pltpu.VMEM((B,), jnp.int32),
        pltpu.VMEM((B, LANES), jnp.float32),
        pltpu.SemaphoreType.DMA,
    ]
    if tail:
        scratch += [
            pltpu.VMEM((tail, d), jnp.float32),
            pltpu.VMEM((tail,), jnp.int32),
            pltpu.VMEM((tail,), jnp.int32),
            pltpu.VMEM((tail, LANES), jnp.float32),
        ]

    @functools.partial(
        pl.kernel,
        mesh=mesh,
        compiler_params=_sc_compiler_params(),
        out_type=jax.ShapeDtypeStruct((kf_total, npad, d), jnp.float32),
        scratch_types=scratch,
    )
    def k(m_hbm, row_hbm, col_hbm, lvx_hbm, out_hbm, acc, gath, ridx, cidx,
          lvb, sem, *tbufs):
        c = lax.axis_index("c")
        s = lax.axis_index("s")
        base = s * es

        for t in range(sweeps):
            kf = c * sweeps + t
            _zero_fill(gath, zrows, d)
            for tz in range(rows_per_sub // zrows):
                pltpu.sync_copy(
                    gath.at[pl.ds(0, zrows)],
                    acc.at[pl.ds(s * rows_per_sub + tz * zrows, zrows)])
            plsc.subcore_barrier()

            @pl.loop(0, n_full)
            def _(i):
                off = base + i * B
                h1 = pltpu.async_copy(row_hbm.at[pl.ds(off, B)], ridx, sem)
                h2 = pltpu.async_copy(col_hbm.at[pl.ds(off, B)], cidx, sem)
                h3 = pltpu.async_copy(lvx_hbm.at[pl.ds(off, B)], lvb, sem)
                h1.wait()
                h2.wait()
                h3.wait()

                @pl.loop(0, B // LANES)
                def _(g):
                    sl = pl.ds(g * LANES, LANES)
                    cidx[sl] = cidx[sl] + jnp.full((LANES,), kf * n, jnp.int32)

                pltpu.sync_copy(m_hbm.at[cidx], gath)
                _scale_rows(gath, lvb, B, d)
                pltpu.sync_copy(gath, acc.at[ridx], add=True)

            if tail:
                gath_t, ridx_t, cidx_t, lvb_t = tbufs
                off = base + n_full * B
                pltpu.sync_copy(row_hbm.at[pl.ds(off, tail)], ridx_t)
                pltpu.sync_copy(col_hbm.at[pl.ds(off, tail)], cidx_t)
                pltpu.sync_copy(lvx_hbm.at[pl.ds(off, tail)], lvb_t)

                @pl.loop(0, tail // LANES)
                def _(g):
                    sl = pl.ds(g * LANES, LANES)
                    cidx_t[sl] = cidx_t[sl] + jnp.full((LANES,), kf * n,
                                                       jnp.int32)

                pltpu.sync_copy(m_hbm.at[cidx_t], gath_t)
                _scale_rows(gath_t, lvb_t, tail, d)
                pltpu.sync_copy(gath_t, acc.at[ridx_t], add=True)

            plsc.subcore_barrier()
            for tz in range(rows_per_sub // zrows):
                r0 = s * rows_per_sub + tz * zrows
                pltpu.sync_copy(acc.at[pl.ds(r0, zrows)],
                                out_hbm.at[kf, pl.ds(r0, zrows)])
            plsc.subcore_barrier()

    return k


@functools.lru_cache(maxsize=None)
def _weights_tc(n, d, kf, p_ord, nb):
    def body(cp0_ref, cp1_ref, eig_ref, a2_ref, m_ref):
        csum = cp0_ref[...] + cp1_ref[...]
        ex = jnp.exp(-eig_ref[...])
        pw = jnp.ones_like(ex)
        w = [None] * kf
        for pp in range(p_ord):
            for kk in range(kf):
                term = pw * a2_ref[kk, pp, :][None, :]
                w[kk] = term if pp == 0 else w[kk] + term
            pw = pw * ex
        for kk in range(kf):
            m_ref[kk] = csum * w[kk]

    return pl.pallas_call(
        body,
        grid=(n // nb,),
        in_specs=[
            pl.BlockSpec((nb, d), lambda i: (i, 0)),
            pl.BlockSpec((nb, d), lambda i: (i, 0)),
            pl.BlockSpec((nb, 1), lambda i: (i, 0)),
            pl.BlockSpec((kf, p_ord, d), lambda i: (0, 0, 0)),
        ],
        out_specs=pl.BlockSpec((kf, nb, d), lambda i: (0, i, 0)),
        out_shape=jax.ShapeDtypeStruct((kf, n, d), jnp.float32),
    )


def kernel(x, L_i, L_v, node_attr_eig, alpha):
    n, d = x.shape
    e = L_v.shape[0]
    _, kf, p_ord = alpha.shape
    row = L_i[0].astype(jnp.int32)
    col = L_i[1].astype(jnp.int32)
    lvx = jnp.broadcast_to(L_v.astype(jnp.float32)[:, None], (e, LANES))

    c_part = _spmm_coeffs(n, d, e)(x, row, col, lvx)[:, :n]
    a2 = jnp.transpose(alpha, (1, 2, 0))
    m = _weights_tc(n, d, kf, p_ord, 1000)(
        c_part[0], c_part[1], node_attr_eig.reshape(n, 1), a2)
    outk = _spmm_out(n, d, e, kf)(m.reshape(kf * n, d), row, col, lvx)
    return jnp.transpose(outk[:, :n], (1, 2, 0))


# speedup vs baseline: 4.9392x; 1.1325x over previous
import dataclasses
import functools

import jax
import jax.numpy as jnp
from jax import lax
from jax.experimental import pallas as pl
from jax.experimental.pallas import tpu as pltpu
from jax.experimental.pallas import tpu_sc as plsc

NC = 2
NS = 16
LANES = 16
B = 96


def _split_chunks(per_worker):
    n_full = per_worker // B
    tail = per_worker - n_full * B
    if tail % 8:
        raise ValueError(f"edge split {per_worker} not 8-aligned with B={B}")
    return n_full, tail


def _sc_compiler_params():
    cp = pltpu.CompilerParams()
    if "needs_layout_passes" in pltpu.CompilerParams.__dataclass_fields__:
        cp = dataclasses.replace(cp, needs_layout_passes=False)
    return cp


def _scale_rows(gath, lvb, n_rows, d):
    @pl.loop(0, n_rows)
    def _(r):
        lvs = lvb[r]
        for j in range(d // LANES):
            sl = pl.ds(j * LANES, LANES)
            gath[r, sl] = gath[r, sl] * lvs


def _zero_fill(zbuf, zrows, d):
    @pl.loop(0, zrows)
    def _(i):
        for j in range(d // LANES):
            zbuf[i, pl.ds(j * LANES, LANES)] = jnp.zeros((LANES,), jnp.float32)


def _pad_rows(n):
    return ((n + NS * 128 - 1) // (NS * 128)) * (NS * 128)


@functools.lru_cache(maxsize=None)
def _spmm_coeffs(n, d, e):
    ew = e // (NC * NS)
    n_full, tail = _split_chunks(ew)
    npad = _pad_rows(n)
    rows_per_sub = npad // NS
    zrows = 64
    mesh = plsc.VectorSubcoreMesh(core_axis_name="c", subcore_axis_name="s")

    scratch = [
        pltpu.VMEM_SHARED((npad, d), jnp.float32),
        pltpu.VMEM((B, d), jnp.float32),
        pltpu.VMEM((B,), jnp.int32),
        pltpu.VMEM((B,), jnp.int32),
        pltpu.VMEM((B, LANES), jnp.float32),
        pltpu.SemaphoreType.DMA,
        pltpu.VMEM((B,), jnp.int32),
        pltpu.VMEM((B,), jnp.int32),
        pltpu.VMEM((B, LANES), jnp.float32),
    ]
    if tail:
        scratch += [
            pltpu.VMEM((tail, d), jnp.float32),
            pltpu.VMEM((tail,), jnp.int32),
            pltpu.VMEM((tail,), jnp.int32),
            pltpu.VMEM((tail, LANES), jnp.float32),
        ]

    @functools.partial(
        pl.kernel,
        mesh=mesh,
        compiler_params=_sc_compiler_params(),
        out_type=jax.ShapeDtypeStruct((NC, npad, d), jnp.float32),
        scratch_types=scratch,
    )
    def k(x_hbm, row_hbm, col_hbm, lvx_hbm, out_hbm, acc, gath, ridx, cidx,
          lvb, sem, ridx2, cidx2, lvb2, *tbufs):
        c = lax.axis_index("c")
        s = lax.axis_index("s")
        wid = s * NC + c
        _zero_fill(gath, zrows, d)
        for t in range(rows_per_sub // zrows):
            pltpu.sync_copy(gath.at[pl.ds(0, zrows)],
                            acc.at[pl.ds(s * rows_per_sub + t * zrows,
                                         zrows)])
        plsc.subcore_barrier()

        base = wid * ew
        set_a = (ridx, cidx, lvb)
        set_b = (ridx2, cidx2, lvb2)

        def issue(bufs, i):
            off = base + i * B
            pltpu.async_copy(row_hbm.at[pl.ds(off, B)], bufs[0], sem)
            pltpu.async_copy(col_hbm.at[pl.ds(off, B)], bufs[1], sem)
            pltpu.async_copy(lvx_hbm.at[pl.ds(off, B)], bufs[2], sem)

        def drain(bufs):
            pltpu.make_async_copy(row_hbm.at[pl.ds(0, B)], bufs[0], sem).wait()
            pltpu.make_async_copy(col_hbm.at[pl.ds(0, B)], bufs[1], sem).wait()
            pltpu.make_async_copy(lvx_hbm.at[pl.ds(0, B)], bufs[2], sem).wait()

        start = n_full % 2
        if start:
            pltpu.sync_copy(row_hbm.at[pl.ds(base, B)], ridx)
            pltpu.sync_copy(col_hbm.at[pl.ds(base, B)], cidx)
            pltpu.sync_copy(lvx_hbm.at[pl.ds(base, B)], lvb)
            pltpu.sync_copy(x_hbm.at[ridx], gath)
            _scale_rows(gath, lvb, B, d)
            pltpu.sync_copy(gath, acc.at[cidx], add=True)
        if n_full > start:
            issue(set_a, start)

            @pl.loop(0, (n_full - start) // 2)
            def _(j):
                for p, (cur, nxt) in enumerate(((set_a, set_b),
                                                (set_b, set_a))):
                    i = start + j * 2 + p
                    drain(cur)
                    issue(nxt, jnp.minimum(i + 1, n_full - 1))
                    pltpu.sync_copy(x_hbm.at[cur[0]], gath)
                    _scale_rows(gath, cur[2], B, d)
                    pltpu.sync_copy(gath, acc.at[cur[1]], add=True)

            drain(set_a)

        if tail:
            gath_t, ridx_t, cidx_t, lvb_t = tbufs
            off = base + n_full * B
            pltpu.sync_copy(row_hbm.at[pl.ds(off, tail)], ridx_t)
            pltpu.sync_copy(col_hbm.at[pl.ds(off, tail)], cidx_t)
            pltpu.sync_copy(lvx_hbm.at[pl.ds(off, tail)], lvb_t)
            pltpu.sync_copy(x_hbm.at[ridx_t], gath_t)
            _scale_rows(gath_t, lvb_t, tail, d)
            pltpu.sync_copy(gath_t, acc.at[cidx_t], add=True)

        plsc.subcore_barrier()
        for t in range(rows_per_sub // zrows):
            r0 = s * rows_per_sub + t * zrows
            pltpu.sync_copy(acc.at[pl.ds(r0, zrows)],
                            out_hbm.at[c, pl.ds(r0, zrows)])

    return k


@functools.lru_cache(maxsize=None)
def _spmm_out(n, d, e, kf_total):
    es = e // NS
    n_full, tail = _split_chunks(es)
    npad = _pad_rows(n)
    rows_per_sub = npad // NS
    zrows = 64
    sweeps = kf_total // NC
    mesh = plsc.VectorSubcoreMesh(core_axis_name="c", subcore_axis_name="s")

    scratch = [
        pltpu.VMEM_SHARED((npad, d), jnp.float32),
        pltpu.VMEM((B, d), jnp.float32),
        pltpu.VMEM((B,), jnp.int32),
        pltpu.VMEM((B,), jnp.int32),
        pltpu.VMEM((B, LANES), jnp.float32),
        pltpu.SemaphoreType.DMA,
        pltpu.VMEM((B,), jnp.int32),
        pltpu.VMEM((B,), jnp.int32),
        pltpu.VMEM((B, LANES), jnp.float32),
    ]
    if tail:
        scratch += [
            pltpu.VMEM((tail, d), jnp.float32),
            pltpu.VMEM((tail,), jnp.int32),
            pltpu.VMEM((tail,), jnp.int32),
            pltpu.VMEM((tail, LANES), jnp.float32),
        ]

    @functools.partial(
        pl.kernel,
        mesh=mesh,
        compiler_params=_sc_compiler_params(),
        out_type=jax.ShapeDtypeStruct((kf_total, npad, d), jnp.float32),
        scratch_types=scratch,
    )
    def k(m_hbm, row_hbm, col_hbm, lvx_hbm, out_hbm, acc, gath, ridx, cidx,
          lvb, sem, ridx2, cidx2, lvb2, *tbufs):
        c = lax.axis_index("c")
        s = lax.axis_index("s")
        base = s * es
        set_a = (ridx, cidx, lvb)
        set_b = (ridx2, cidx2, lvb2)

        def issue(bufs, i):
            off = base + i * B
            pltpu.async_copy(row_hbm.at[pl.ds(off, B)], bufs[0], sem)
            pltpu.async_copy(col_hbm.at[pl.ds(off, B)], bufs[1], sem)
            pltpu.async_copy(lvx_hbm.at[pl.ds(off, B)], bufs[2], sem)

        def drain(bufs):
            pltpu.make_async_copy(row_hbm.at[pl.ds(0, B)], bufs[0], sem).wait()
            pltpu.make_async_copy(col_hbm.at[pl.ds(0, B)], bufs[1], sem).wait()
            pltpu.make_async_copy(lvx_hbm.at[pl.ds(0, B)], bufs[2], sem).wait()

        def consume(cur, kf):
            @pl.loop(0, B // LANES)
            def _(g):
                sl = pl.ds(g * LANES, LANES)
                cur[1][sl] = cur[1][sl] + jnp.full((LANES,), kf * n,
                                                   jnp.int32)

            pltpu.sync_copy(m_hbm.at[cur[1]], gath)
            _scale_rows(gath, cur[2], B, d)
            pltpu.sync_copy(gath, acc.at[cur[0]], add=True)

        start = n_full % 2
        for t in range(sweeps):
            kf = c * sweeps + t
            _zero_fill(gath, zrows, d)
            for tz in range(rows_per_sub // zrows):
                pltpu.sync_copy(
                    gath.at[pl.ds(0, zrows)],
                    acc.at[pl.ds(s * rows_per_sub + tz * zrows, zrows)])
            plsc.subcore_barrier()

            if start:
                pltpu.sync_copy(row_hbm.at[pl.ds(base, B)], ridx)
                pltpu.sync_copy(col_hbm.at[pl.ds(base, B)], cidx)
                pltpu.sync_copy(lvx_hbm.at[pl.ds(base, B)], lvb)
                consume(set_a, kf)
            if n_full > start:
                issue(set_a, start)

                @pl.loop(0, (n_full - start) // 2)
                def _(j):
                    for p, (cur, nxt) in enumerate(((set_a, set_b),
                                                    (set_b, set_a))):
                        i = start + j * 2 + p
                        drain(cur)
                        issue(nxt, jnp.minimum(i + 1, n_full - 1))
                        consume(cur, kf)

                drain(set_a)

            if tail:
                gath_t, ridx_t, cidx_t, lvb_t = tbufs
                off = base + n_full * B
                pltpu.sync_copy(row_hbm.at[pl.ds(off, tail)], ridx_t)
                pltpu.sync_copy(col_hbm.at[pl.ds(off, tail)], cidx_t)
                pltpu.sync_copy(lvx_hbm.at[pl.ds(off, tail)], lvb_t)

                @pl.loop(0, tail // LANES)
                def _(g):
                    sl = pl.ds(g * LANES, LANES)
                    cidx_t[sl] = cidx_t[sl] + jnp.full((LANES,), kf * n,
                                                       jnp.int32)

                pltpu.sync_copy(m_hbm.at[cidx_t], gath_t)
                _scale_rows(gath_t, lvb_t, tail, d)
                pltpu.sync_copy(gath_t, acc.at[ridx_t], add=True)

            plsc.subcore_barrier()
            for tz in range(rows_per_sub // zrows):
                r0 = s * rows_per_sub + tz * zrows
                pltpu.sync_copy(acc.at[pl.ds(r0, zrows)],
                                out_hbm.at[kf, pl.ds(r0, zrows)])
            plsc.subcore_barrier()

    return k


@functools.lru_cache(maxsize=None)
def _weights_tc(n, d, kf, p_ord, nb):
    def body(cp0_ref, cp1_ref, eig_ref, a2_ref, m_ref):
        csum = cp0_ref[...] + cp1_ref[...]
        ex = jnp.exp(-eig_ref[...])
        pw = jnp.ones_like(ex)
        w = [None] * kf
        for pp in range(p_ord):
            for kk in range(kf):
                term = pw * a2_ref[kk, pp, :][None, :]
                w[kk] = term if pp == 0 else w[kk] + term
            pw = pw * ex
        for kk in range(kf):
            m_ref[kk] = csum * w[kk]

    return pl.pallas_call(
        body,
        grid=(n // nb,),
        in_specs=[
            pl.BlockSpec((nb, d), lambda i: (i, 0)),
            pl.BlockSpec((nb, d), lambda i: (i, 0)),
            pl.BlockSpec((nb, 1), lambda i: (i, 0)),
            pl.BlockSpec((kf, p_ord, d), lambda i: (0, 0, 0)),
        ],
        out_specs=pl.BlockSpec((kf, nb, d), lambda i: (0, i, 0)),
        out_shape=jax.ShapeDtypeStruct((kf, n, d), jnp.float32),
    )


def kernel(x, L_i, L_v, node_attr_eig, alpha):
    n, d = x.shape
    e = L_v.shape[0]
    _, kf, p_ord = alpha.shape
    row = L_i[0].astype(jnp.int32)
    col = L_i[1].astype(jnp.int32)
    lvx = jnp.broadcast_to(L_v.astype(jnp.float32)[:, None], (e, LANES))

    c_part = _spmm_coeffs(n, d, e)(x, row, col, lvx)[:, :n]
    a2 = jnp.transpose(alpha, (1, 2, 0))
    m = _weights_tc(n, d, kf, p_ord, 1000)(
        c_part[0], c_part[1], node_attr_eig.reshape(n, 1), a2)
    outk = _spmm_out(n, d, e, kf)(m.reshape(kf * n, d), row, col, lvx)
    return jnp.transpose(outk[:, :n], (1, 2, 0))
